# baseline (speedup 1.0000x reference)
import jax
import jax.numpy as jnp
from jax.experimental import pallas as pl

_N_CODES = 512
_CODE_DIM = 32
_COMMITMENT = 0.25
_ROWS = 64 * 1024
_BLOCK = 1024
_GRID = _ROWS // _BLOCK


def _vq_body(z_ref, e_ref, ehi_ref, elo_ref, tri_ref, wi_ref,
             zsq_ref, esq_ref, zq_ref, idx_ref, loss_ref):
    i = pl.program_id(0)
    z = z_ref[...]
    e = e_ref[...]
    scores = jax.lax.dot_general(
        z.astype(jnp.bfloat16), e.astype(jnp.bfloat16), (((1,), (1,)), ((), ())),
        preferred_element_type=jnp.float32)
    base = zsq_ref[...] + esq_ref[...]
    dist = base - 2.0 * scores
    m = jnp.min(dist, axis=1, keepdims=True)
    mask = dist == m
    maskbf = mask.astype(jnp.bfloat16)
    t = jax.lax.dot_general(
        maskbf, tri_ref[...], (((1,), (0,)), ((), ())),
        preferred_element_type=jnp.float32)
    u = jnp.where(t == 1.0, maskbf, jnp.bfloat16(0.0))
    zq = (jax.lax.dot_general(
              u, ehi_ref[...], (((1,), (0,)), ((), ())),
              preferred_element_type=jnp.float32)
          + jax.lax.dot_general(
              u, elo_ref[...], (((1,), (0,)), ((), ())),
              preferred_element_type=jnp.float32))
    iw = jax.lax.dot_general(
        u, wi_ref[...], (((1,), (0,)), ((), ())),
        preferred_element_type=jnp.float32)
    idx_ref[...] = (iw[:, 0:1] + iw[:, 1:2]).astype(jnp.int32)
    zq_ref[...] = z + (zq - z)
    diff = zq - z

    @pl.when(i == 0)
    def _init():
        loss_ref[...] = jnp.zeros_like(loss_ref)

    loss_ref[...] += jnp.sum(diff * diff, axis=0, keepdims=True)


@jax.jit
def _vq(zf, embedding, ehi, elo, tri, wi, zsq, esq):
    zq, idx, loss = pl.pallas_call(
        _vq_body,
        grid=(_GRID,),
        in_specs=[
            pl.BlockSpec((_BLOCK, _CODE_DIM), lambda i: (i, 0)),
            pl.BlockSpec((_N_CODES, _CODE_DIM), lambda i: (0, 0)),
            pl.BlockSpec((_N_CODES, _CODE_DIM), lambda i: (0, 0)),
            pl.BlockSpec((_N_CODES, _CODE_DIM), lambda i: (0, 0)),
            pl.BlockSpec((_N_CODES, _N_CODES), lambda i: (0, 0)),
            pl.BlockSpec((_N_CODES, 8), lambda i: (0, 0)),
            pl.BlockSpec((_BLOCK, 1), lambda i: (i, 0)),
            pl.BlockSpec((1, _N_CODES), lambda i: (0, 0)),
        ],
        out_specs=[
            pl.BlockSpec((_BLOCK, _CODE_DIM), lambda i: (i, 0)),
            pl.BlockSpec((_BLOCK, 1), lambda i: (i, 0)),
            pl.BlockSpec((1, _CODE_DIM), lambda i: (0, 0)),
        ],
        out_shape=[
            jax.ShapeDtypeStruct((_ROWS, _CODE_DIM), jnp.float32),
            jax.ShapeDtypeStruct((_ROWS, 1), jnp.int32),
            jax.ShapeDtypeStruct((1, _CODE_DIM), jnp.float32),
        ],
    )(zf, embedding, ehi, elo, tri, wi, zsq, esq)
    return zq, idx, loss


def kernel(z, embedding):
    b, n, d = z.shape
    zf = z.reshape(b * n, d)
    zsq = jnp.sum(zf ** 2, axis=-1, keepdims=True)
    esq = jnp.sum(embedding ** 2, axis=-1)[None, :]
    ehi = embedding.astype(jnp.bfloat16)
    elo = (embedding - ehi.astype(jnp.float32)).astype(jnp.bfloat16)
    k = jnp.arange(_N_CODES)
    tri = (k[:, None] <= k[None, :]).astype(jnp.bfloat16)
    wi = jnp.zeros((_N_CODES, 8), jnp.float32)
    wi = wi.at[:, 0].set((k % 256).astype(jnp.float32))
    wi = wi.at[:, 1].set((k // 256 * 256).astype(jnp.float32))
    wi = wi.astype(jnp.bfloat16)
    zq, idx, loss = _vq(zf, embedding, ehi, elo, tri, wi, zsq, esq)
    vq_loss = jnp.sum(loss) * ((1.0 + _COMMITMENT) / (b * n * d))
    return zq.reshape(b, n, d), idx.reshape(b, n), vq_loss


# speedup vs baseline: 1.0451x; 1.0451x over previous
import jax
import jax.numpy as jnp
from jax.experimental import pallas as pl

_N_CODES = 512
_CODE_DIM = 32
_COMMITMENT = 0.25
_ROWS = 64 * 1024
_BLOCK = 1024
_GRID = _ROWS // _BLOCK


def _vq_body(z_ref, e_ref, ehi_ref, elo_ref, tri_ref, wi_ref,
             zsq_ref, esq_ref, zq_ref, idx_ref, loss_ref):
    i = pl.program_id(0)
    z = z_ref[...]
    e = e_ref[...]
    scores = jax.lax.dot_general(
        z.astype(jnp.bfloat16), e.astype(jnp.bfloat16), (((1,), (1,)), ((), ())),
        preferred_element_type=jnp.float32)
    base = zsq_ref[...] + esq_ref[...]
    dist = base - 2.0 * scores
    m = jnp.min(dist, axis=1, keepdims=True)
    mask = dist == m
    maskbf = mask.astype(jnp.bfloat16)
    t = jax.lax.dot_general(
        maskbf, tri_ref[...], (((1,), (0,)), ((), ())),
        preferred_element_type=jnp.float32)
    u = jnp.where(t == 1.0, maskbf, jnp.bfloat16(0.0))
    zq = (jax.lax.dot_general(
              u, ehi_ref[...], (((1,), (0,)), ((), ())),
              preferred_element_type=jnp.float32)
          + jax.lax.dot_general(
              u, elo_ref[...], (((1,), (0,)), ((), ())),
              preferred_element_type=jnp.float32))
    iw = jax.lax.dot_general(
        wi_ref[...], u, (((0,), (1,)), ((), ())),
        preferred_element_type=jnp.float32)
    idx_ref[0, :, :] = (iw[0:1, :] + iw[1:2, :]).astype(jnp.int32)
    zq_ref[...] = z + (zq - z)
    diff = zq - z

    @pl.when(i == 0)
    def _init():
        loss_ref[...] = jnp.zeros_like(loss_ref)

    loss_ref[...] += jnp.sum(diff * diff, axis=0, keepdims=True)


@jax.jit
def _vq(zf, embedding, ehi, elo, tri, wi, zsq, esq):
    zq, idx, loss = pl.pallas_call(
        _vq_body,
        grid=(_GRID,),
        in_specs=[
            pl.BlockSpec((_BLOCK, _CODE_DIM), lambda i: (i, 0)),
            pl.BlockSpec((_N_CODES, _CODE_DIM), lambda i: (0, 0)),
            pl.BlockSpec((_N_CODES, _CODE_DIM), lambda i: (0, 0)),
            pl.BlockSpec((_N_CODES, _CODE_DIM), lambda i: (0, 0)),
            pl.BlockSpec((_N_CODES, _N_CODES), lambda i: (0, 0)),
            pl.BlockSpec((_N_CODES, 8), lambda i: (0, 0)),
            pl.BlockSpec((_BLOCK, 1), lambda i: (i, 0)),
            pl.BlockSpec((1, _N_CODES), lambda i: (0, 0)),
        ],
        out_specs=[
            pl.BlockSpec((_BLOCK, _CODE_DIM), lambda i: (i, 0)),
            pl.BlockSpec((1, 1, _BLOCK), lambda i: (i, 0, 0)),
            pl.BlockSpec((1, _CODE_DIM), lambda i: (0, 0)),
        ],
        out_shape=[
            jax.ShapeDtypeStruct((_ROWS, _CODE_DIM), jnp.float32),
            jax.ShapeDtypeStruct((_GRID, 1, _BLOCK), jnp.int32),
            jax.ShapeDtypeStruct((1, _CODE_DIM), jnp.float32),
        ],
    )(zf, embedding, ehi, elo, tri, wi, zsq, esq)
    return zq, idx, loss


def kernel(z, embedding):
    b, n, d = z.shape
    zf = z.reshape(b * n, d)
    zsq = jnp.sum(zf ** 2, axis=-1, keepdims=True)
    esq = jnp.sum(embedding ** 2, axis=-1)[None, :]
    ehi = embedding.astype(jnp.bfloat16)
    elo = (embedding - ehi.astype(jnp.float32)).astype(jnp.bfloat16)
    k = jnp.arange(_N_CODES)
    tri = (k[:, None] <= k[None, :]).astype(jnp.bfloat16)
    wi = jnp.zeros((_N_CODES, 8), jnp.float32)
    wi = wi.at[:, 0].set((k % 256).astype(jnp.float32))
    wi = wi.at[:, 1].set((k // 256 * 256).astype(jnp.float32))
    wi = wi.astype(jnp.bfloat16)
    zq, idx, loss = _vq(zf, embedding, ehi, elo, tri, wi, zsq, esq)
    vq_loss = jnp.sum(loss) * ((1.0 + _COMMITMENT) / (b * n * d))
    return zq.reshape(b, n, d), idx.reshape(b, n), vq_loss


# speedup vs baseline: 1.1180x; 1.0697x over previous
import functools

import jax
import jax.numpy as jnp
from jax import lax
from jax.experimental import pallas as pl
from jax.experimental.pallas import tpu as pltpu
from jax.experimental.pallas import tpu_sc as plsc

_N_CODES = 512
_CODE_DIM = 32
_COMMITMENT = 0.25
_ROWS = 64 * 1024
_BLOCK = 1024
_GRID = _ROWS // _BLOCK

_NW = 32
_B_PER_W = _ROWS // _NW
_CHUNK = 128
_NCHUNK = _B_PER_W // _CHUNK


def _vq_body(z_ref, e_ref, zsq_ref, esq_ref, idx_ref, loss_ref):
    i = pl.program_id(0)
    z = z_ref[...]
    e = e_ref[...]
    scores = jax.lax.dot_general(
        z.astype(jnp.bfloat16), e.astype(jnp.bfloat16), (((1,), (1,)), ((), ())),
        preferred_element_type=jnp.float32)
    base = zsq_ref[...] + esq_ref[...]
    dist = base - 2.0 * scores
    m = jnp.min(dist, axis=1, keepdims=True)
    mask = dist == m
    iota = jax.lax.broadcasted_iota(jnp.int32, (_BLOCK, _N_CODES), 1)
    idx = jnp.min(jnp.where(mask, iota, _N_CODES), axis=1).astype(jnp.int32)
    idx_ref[0, 0, :] = idx

    @pl.when(i == 0)
    def _init():
        loss_ref[...] = jnp.zeros_like(loss_ref)

    loss_ref[...] += jnp.sum(jnp.where(mask, dist, 0.0), axis=0, keepdims=True)


_QROWS = 512
_NQ = _B_PER_W // _QROWS


@functools.partial(
    pl.kernel,
    mesh=plsc.VectorSubcoreMesh(core_axis_name="c", subcore_axis_name="s"),
    out_type=jax.ShapeDtypeStruct((_ROWS, 128), jnp.float32),
    scratch_types=[
        pltpu.VMEM((_NCHUNK, _CHUNK), jnp.int32),
        pltpu.VMEM((_QROWS, 128), jnp.float32),
        pltpu.SemaphoreType.DMA,
    ],
)
def _sc_gather(table_hbm, idx_hbm, out_hbm, idx_v, rows_v, sem):
    wid = lax.axis_index("s") * 2 + lax.axis_index("c")
    base = wid * _B_PER_W
    pltpu.sync_copy(idx_hbm.at[wid], idx_v)
    for q in range(_NQ):
        copies = []
        for jj in range(_QROWS // _CHUNK):
            j = q * (_QROWS // _CHUNK) + jj
            copies.append(pltpu.async_copy(
                table_hbm.at[idx_v.at[j]],
                rows_v.at[pl.ds(jj * _CHUNK, _CHUNK)], sem))
        for c in copies:
            c.wait()
        pltpu.sync_copy(rows_v,
                        out_hbm.at[pl.ds(base + q * _QROWS, _QROWS)])


@jax.jit
def _vq(zf, embedding, zsq, esq):
    idx, loss = pl.pallas_call(
        _vq_body,
        grid=(_GRID,),
        in_specs=[
            pl.BlockSpec((_BLOCK, _CODE_DIM), lambda i: (i, 0)),
            pl.BlockSpec((_N_CODES, _CODE_DIM), lambda i: (0, 0)),
            pl.BlockSpec((_BLOCK, 1), lambda i: (i, 0)),
            pl.BlockSpec((1, _N_CODES), lambda i: (0, 0)),
        ],
        out_specs=[
            pl.BlockSpec((1, 1, _BLOCK), lambda i: (i, 0, 0)),
            pl.BlockSpec((1, _N_CODES), lambda i: (0, 0)),
        ],
        out_shape=[
            jax.ShapeDtypeStruct((_GRID, 1, _BLOCK), jnp.int32),
            jax.ShapeDtypeStruct((1, _N_CODES), jnp.float32),
        ],
    )(zf, embedding, zsq, esq)
    return idx, loss


def kernel(z, embedding):
    b, n, d = z.shape
    zf = z.reshape(b * n, d)
    zsq = jnp.sum(zf ** 2, axis=-1, keepdims=True)
    esq = jnp.sum(embedding ** 2, axis=-1)[None, :]
    idx, loss = _vq(zf, embedding, zsq, esq)
    idx3 = idx.reshape(_NW, _NCHUNK, _CHUNK)
    epad = jnp.pad(embedding, ((0, 0), (0, 128 - _CODE_DIM)))
    zq = _sc_gather(epad, idx3)[:, :_CODE_DIM]
    vq_loss = jnp.sum(loss) * ((1.0 + _COMMITMENT) / (b * n * d))
    return zq.reshape(b, n, d), idx.reshape(b, n), vq_loss
